# baseline (speedup 1.0000x reference)
import functools

import jax
import jax.numpy as jnp
import numpy as np
from jax import lax
from jax.experimental import pallas as pl
from jax.experimental.pallas import tpu as pltpu
from jax.experimental.pallas import tpu_sc as plsc

_ALPHA = 0.5
_TAU = 0.1
_KS = 128
_NC = 2
_NS = 16
_NW = _NC * _NS
_LANES = 16
_BW = 2048


def _sampling_constants(n: int):
    with jax.ensure_compile_time_eval():
        cpu = jax.local_devices(backend="cpu")[0]
        with jax.default_device(cpu):
            j_vals = jnp.arange(n, dtype=jnp.float32)
            log_probs = -(1.0 + _ALPHA - _TAU) * jnp.log(n - j_vals + 1e-08)
            probs = jnp.exp(log_probs - jax.nn.logsumexp(log_probs))
            idx = jax.random.choice(jax.random.key(1), n, shape=(_KS,),
                                    replace=False, p=probs)
            idx = idx.astype(jnp.int32)
            jf = idx.astype(jnp.float32)
            true_w = jnp.power(n - jf + 1e-08, -(1.0 + _ALPHA))
            samp_p = jnp.power(n - jf + 1e-08, -(1.0 + _ALPHA - _TAU))
            w = true_w / (samp_p + 1e-08)
            hist = (n - 1 - idx).astype(jnp.int32)
            return np.asarray(hist, np.int32), np.asarray(w, np.float32)


_CONST_CACHE = {}


def _consts(n: int):
    if n not in _CONST_CACHE:
        _CONST_CACHE[n] = _sampling_constants(n)
    return _CONST_CACHE[n]


_CS = 512


def _chunk_plan(n: int):
    hist, w = _consts(n)
    nsub = n // _CS
    subs = sorted(set(int(h) // _CS for h in hist) | {nsub - 1})
    pos = {q: i for i, q in enumerate(subs)}
    remap = np.array([pos[int(h) // _CS] * _CS + int(h) % _CS for h in hist],
                     dtype=np.int32)
    cur_at = pos[nsub - 1] * _CS + (_CS - 1 - ((n - 1) % _CS) % _CS)
    cur_at = pos[nsub - 1] * _CS + ((n - 1) % _CS)
    packed = np.concatenate([remap, w.view(np.int32)]).astype(np.int32)
    return subs, cur_at, packed, float(w.sum())


def _make_sc_reduce(b: int, n: int, subs, cur_at: int, wsum: float):
    rpw = b // _NW
    nsr = len(subs)
    per_row = nsr * _CS
    nsub = n // _CS
    mesh = plsc.VectorSubcoreMesh(core_axis_name="c", subcore_axis_name="s",
                                  num_cores=_NC, num_subcores=_NS)

    @functools.partial(
        pl.kernel,
        out_type=jax.ShapeDtypeStruct((b, _LANES), jnp.float32),
        mesh=mesh,
        scratch_types=[
            pltpu.VMEM((2 * _KS,), jnp.int32),
            pltpu.VMEM((rpw * per_row,), jnp.float32),
            pltpu.VMEM((_LANES,), jnp.float32),
            pltpu.SemaphoreType.DMA,
        ],
        compiler_params=pltpu.CompilerParams(needs_layout_passes=False),
    )
    def sc_reduce(x3, consts_hbm, out, cv, rows_v, res_v, sem):
        cid = lax.axis_index("c")
        sid = lax.axis_index("s")
        wid = sid * _NC + cid
        row0 = wid * rpw
        cps = []
        for rl in range(rpw):
            base = rl * per_row
            sub0 = (row0 + rl) * nsub
            for i, q in enumerate(subs):
                cps.append(pltpu.async_copy(
                    x3.at[sub0 + q],
                    rows_v.at[pl.ds(base + i * _CS, _CS)], sem))
        pltpu.sync_copy(consts_hbm, cv)
        for cp in cps:
            cp.wait()
        lane = lax.iota(jnp.int32, _LANES)
        for rl in range(rpw):
            base = rl * per_row
            acc = jnp.zeros((_LANES,), jnp.float32)
            for j in range(_KS // _LANES):
                wv = plsc.bitcast(cv[pl.ds(_KS + j * _LANES, _LANES)],
                                  jnp.float32)
                idx16 = cv[pl.ds(j * _LANES, _LANES)] + base
                vals16 = plsc.load_gather(rows_v, [idx16])
                acc = acc + wv * vals16
            dot = jnp.sum(acc)
            curbase = base + (cur_at // _LANES) * _LANES
            cur = rows_v[pl.ds(curbase, _LANES)][cur_at % _LANES]
            res = (cur * wsum - dot) * (1.0 / _KS)
            res_v[...] = jnp.where(lane == _LANES - 1, res, 0.0)
            pltpu.sync_copy(res_v, out.at[row0 + rl])

    return sc_reduce


def _tc_zeros_body(o_ref):
    o_ref[...] = jnp.zeros_like(o_ref)


def _make_tc_zeros(b: int, n: int):
    return pl.pallas_call(
        _tc_zeros_body,
        grid=(n // _BW,),
        out_specs=pl.BlockSpec((b, _BW), lambda j: (0, j)),
        out_shape=jax.ShapeDtypeStruct((b, n), jnp.float32),
    )


def kernel(x):
    b, n = x.shape
    subs, cur_at, packed, wsum = _chunk_plan(n)
    x3 = x.reshape(b * (n // _CS), _CS)
    upd = _make_sc_reduce(b, n, tuple(subs), cur_at, wsum)(
        x3, jnp.asarray(packed))
    zeros = _make_tc_zeros(b, n)()
    return lax.dynamic_update_slice(zeros, upd, (0, n - _LANES))


# speedup vs baseline: 1.3446x; 1.3446x over previous
import functools

import jax
import jax.numpy as jnp
import numpy as np
from jax import lax
from jax.experimental import pallas as pl
from jax.experimental.pallas import tpu as pltpu
from jax.experimental.pallas import tpu_sc as plsc

_ALPHA = 0.5
_TAU = 0.1
_KS = 128
_NC = 2
_NS = 16
_NW = _NC * _NS
_LANES = 16
_BW = 2048


def _sampling_constants(n: int):
    with jax.ensure_compile_time_eval():
        cpu = jax.local_devices(backend="cpu")[0]
        with jax.default_device(cpu):
            j_vals = jnp.arange(n, dtype=jnp.float32)
            log_probs = -(1.0 + _ALPHA - _TAU) * jnp.log(n - j_vals + 1e-08)
            probs = jnp.exp(log_probs - jax.nn.logsumexp(log_probs))
            idx = jax.random.choice(jax.random.key(1), n, shape=(_KS,),
                                    replace=False, p=probs)
            idx = idx.astype(jnp.int32)
            jf = idx.astype(jnp.float32)
            true_w = jnp.power(n - jf + 1e-08, -(1.0 + _ALPHA))
            samp_p = jnp.power(n - jf + 1e-08, -(1.0 + _ALPHA - _TAU))
            w = true_w / (samp_p + 1e-08)
            hist = (n - 1 - idx).astype(jnp.int32)
            return np.asarray(hist, np.int32), np.asarray(w, np.float32)


_CONST_CACHE = {}


def _consts(n: int):
    if n not in _CONST_CACHE:
        _CONST_CACHE[n] = _sampling_constants(n)
    return _CONST_CACHE[n]


def _make_sc_reduce(b: int, n: int, wsum: float):
    rpw = b // _NW
    mesh = plsc.VectorSubcoreMesh(core_axis_name="c", subcore_axis_name="s",
                                  num_cores=_NC, num_subcores=_NS)

    @functools.partial(
        pl.kernel,
        out_type=jax.ShapeDtypeStruct((b, _LANES), jnp.float32),
        mesh=mesh,
        scratch_types=[
            pltpu.VMEM((2 * _KS,), jnp.int32),
            pltpu.VMEM((rpw * n,), jnp.float32),
            pltpu.VMEM((_LANES,), jnp.float32),
            pltpu.SemaphoreType.DMA,
        ],
        compiler_params=pltpu.CompilerParams(needs_layout_passes=False),
    )
    def sc_reduce(x2, consts_hbm, out, cv, rows_v, res_v, sem):
        cid = lax.axis_index("c")
        sid = lax.axis_index("s")
        wid = sid * _NC + cid
        row0 = wid * rpw
        cps = [
            pltpu.async_copy(x2.at[row0 + rl],
                             rows_v.at[pl.ds(rl * n, n)], sem)
            for rl in range(rpw)
        ]
        pltpu.sync_copy(consts_hbm, cv)
        for cp in cps:
            cp.wait()
        lane = lax.iota(jnp.int32, _LANES)
        for rl in range(rpw):
            acc = jnp.zeros((_LANES,), jnp.float32)
            for j in range(_KS // _LANES):
                wv = plsc.bitcast(cv[pl.ds(_KS + j * _LANES, _LANES)],
                                  jnp.float32)
                idx16 = cv[pl.ds(j * _LANES, _LANES)] + (rl * n)
                vals16 = plsc.load_gather(rows_v, [idx16])
                acc = acc + wv * vals16
            dot = jnp.sum(acc)
            cur = rows_v[pl.ds(rl * n + n - _LANES, _LANES)][_LANES - 1]
            res = (cur * wsum - dot) * (1.0 / _KS)
            res_v[...] = jnp.where(lane == _LANES - 1, res, 0.0)
            pltpu.sync_copy(res_v, out.at[row0 + rl])

    return sc_reduce


def _tc_zeros_body(o_ref, z_ref, sem):
    z_ref[...] = jnp.zeros_like(z_ref)
    nblk = o_ref.shape[1] // _BW
    cps = [
        pltpu.make_async_copy(z_ref, o_ref.at[:, pl.ds(j * _BW, _BW)], sem)
        for j in range(nblk)
    ]
    for cp in cps:
        cp.start()
    for cp in cps:
        cp.wait()


def _make_tc_zeros(b: int, n: int):
    return pl.pallas_call(
        _tc_zeros_body,
        out_specs=pl.BlockSpec(memory_space=pl.ANY),
        out_shape=jax.ShapeDtypeStruct((b, n), jnp.float32),
        scratch_shapes=[
            pltpu.VMEM((b, _BW), jnp.float32),
            pltpu.SemaphoreType.DMA,
        ],
    )


def kernel(x):
    b, n = x.shape
    hist, w = _consts(n)
    packed = np.concatenate([hist, w.view(np.int32)]).astype(np.int32)
    upd = _make_sc_reduce(b, n, float(w.sum()))(x, jnp.asarray(packed))
    zeros = _make_tc_zeros(b, n)()
    return lax.dynamic_update_slice(zeros, upd, (0, n - _LANES))


# speedup vs baseline: 1.3461x; 1.0011x over previous
import functools

import jax
import jax.numpy as jnp
import numpy as np
from jax import lax
from jax.experimental import pallas as pl
from jax.experimental.pallas import tpu as pltpu
from jax.experimental.pallas import tpu_sc as plsc

_ALPHA = 0.5
_TAU = 0.1
_KS = 128
_NC = 2
_NS = 16
_NW = _NC * _NS
_LANES = 16
_BW = 2048


def _sampling_constants(n: int):
    with jax.ensure_compile_time_eval():
        cpu = jax.local_devices(backend="cpu")[0]
        with jax.default_device(cpu):
            j_vals = jnp.arange(n, dtype=jnp.float32)
            log_probs = -(1.0 + _ALPHA - _TAU) * jnp.log(n - j_vals + 1e-08)
            probs = jnp.exp(log_probs - jax.nn.logsumexp(log_probs))
            idx = jax.random.choice(jax.random.key(1), n, shape=(_KS,),
                                    replace=False, p=probs)
            idx = idx.astype(jnp.int32)
            jf = idx.astype(jnp.float32)
            true_w = jnp.power(n - jf + 1e-08, -(1.0 + _ALPHA))
            samp_p = jnp.power(n - jf + 1e-08, -(1.0 + _ALPHA - _TAU))
            w = true_w / (samp_p + 1e-08)
            hist = (n - 1 - idx).astype(jnp.int32)
            return np.asarray(hist, np.int32), np.asarray(w, np.float32)


_CONST_CACHE = {}


def _consts(n: int):
    if n not in _CONST_CACHE:
        _CONST_CACHE[n] = _sampling_constants(n)
    return _CONST_CACHE[n]


def _make_sc_reduce(b: int, n: int, hist, w, wsum: float):
    rpw = b // _NW
    hist_l = [int(v) for v in hist]
    w_l = [float(v) for v in w]
    mesh = plsc.VectorSubcoreMesh(core_axis_name="c", subcore_axis_name="s",
                                  num_cores=_NC, num_subcores=_NS)

    @functools.partial(
        pl.kernel,
        out_type=jax.ShapeDtypeStruct((b, _LANES), jnp.float32),
        mesh=mesh,
        scratch_types=[
            pltpu.VMEM((rpw * n,), jnp.float32),
            pltpu.VMEM((_LANES,), jnp.float32),
            pltpu.SemaphoreType.DMA,
        ],
        compiler_params=pltpu.CompilerParams(needs_layout_passes=False),
    )
    def sc_reduce(x2, out, rows_v, res_v, sem):
        cid = lax.axis_index("c")
        sid = lax.axis_index("s")
        wid = sid * _NC + cid
        row0 = wid * rpw
        cps = [
            pltpu.async_copy(x2.at[row0 + rl],
                             rows_v.at[pl.ds(rl * n, n)], sem)
            for rl in range(rpw)
        ]
        lane = lax.iota(jnp.int32, _LANES)
        ivecs, wvecs = [], []
        for j in range(_KS // _LANES):
            ivv = jnp.zeros((_LANES,), jnp.int32)
            wvv = jnp.zeros((_LANES,), jnp.float32)
            for i in range(_LANES):
                ivv = jnp.where(lane == i, hist_l[j * _LANES + i], ivv)
                wvv = jnp.where(lane == i, w_l[j * _LANES + i], wvv)
            ivecs.append(ivv)
            wvecs.append(wvv)
        for cp in cps:
            cp.wait()
        for rl in range(rpw):
            acc = jnp.zeros((_LANES,), jnp.float32)
            for j in range(_KS // _LANES):
                vals16 = plsc.load_gather(rows_v, [ivecs[j] + (rl * n)])
                acc = acc + wvecs[j] * vals16
            dot = jnp.sum(acc)
            cur = rows_v[pl.ds(rl * n + n - _LANES, _LANES)][_LANES - 1]
            res = (cur * wsum - dot) * (1.0 / _KS)
            res_v[...] = jnp.where(lane == _LANES - 1, res, 0.0)
            pltpu.sync_copy(res_v, out.at[row0 + rl])

    return sc_reduce


def _tc_zeros_body(o_ref, z_ref, sem):
    z_ref[...] = jnp.zeros_like(z_ref)
    nblk = o_ref.shape[1] // _BW
    cps = [
        pltpu.make_async_copy(z_ref, o_ref.at[:, pl.ds(j * _BW, _BW)], sem)
        for j in range(nblk)
    ]
    for cp in cps:
        cp.start()
    for cp in cps:
        cp.wait()


def _make_tc_zeros(b: int, n: int):
    return pl.pallas_call(
        _tc_zeros_body,
        out_specs=pl.BlockSpec(memory_space=pl.ANY),
        out_shape=jax.ShapeDtypeStruct((b, n), jnp.float32),
        scratch_shapes=[
            pltpu.VMEM((b, _BW), jnp.float32),
            pltpu.SemaphoreType.DMA,
        ],
    )


def kernel(x):
    b, n = x.shape
    hist, w = _consts(n)
    upd = _make_sc_reduce(b, n, hist, w, float(w.sum()))(x)
    zeros = _make_tc_zeros(b, n)()
    return lax.dynamic_update_slice(zeros, upd, (0, n - _LANES))


# speedup vs baseline: 1.3524x; 1.0047x over previous
import functools

import jax
import jax.numpy as jnp
import numpy as np
from jax import lax
from jax.experimental import pallas as pl
from jax.experimental.pallas import tpu as pltpu
from jax.experimental.pallas import tpu_sc as plsc

_ALPHA = 0.5
_TAU = 0.1
_KS = 128
_NC = 2
_NS = 16
_NW = _NC * _NS
_LANES = 16
_BW = 2048


def _sampling_constants(n: int):
    with jax.ensure_compile_time_eval():
        cpu = jax.local_devices(backend="cpu")[0]
        with jax.default_device(cpu):
            j_vals = jnp.arange(n, dtype=jnp.float32)
            log_probs = -(1.0 + _ALPHA - _TAU) * jnp.log(n - j_vals + 1e-08)
            probs = jnp.exp(log_probs - jax.nn.logsumexp(log_probs))
            idx = jax.random.choice(jax.random.key(1), n, shape=(_KS,),
                                    replace=False, p=probs)
            idx = idx.astype(jnp.int32)
            jf = idx.astype(jnp.float32)
            true_w = jnp.power(n - jf + 1e-08, -(1.0 + _ALPHA))
            samp_p = jnp.power(n - jf + 1e-08, -(1.0 + _ALPHA - _TAU))
            w = true_w / (samp_p + 1e-08)
            hist = (n - 1 - idx).astype(jnp.int32)
            return np.asarray(hist, np.int32), np.asarray(w, np.float32)


_CONST_CACHE = {}


def _consts(n: int):
    if n not in _CONST_CACHE:
        _CONST_CACHE[n] = _sampling_constants(n)
    return _CONST_CACHE[n]


def _make_sc_reduce(b: int, n: int, hist, w, wsum: float):
    rpw = b // _NW
    hist_l = [int(v) for v in hist]
    w_l = [float(v) for v in w]
    mesh = plsc.VectorSubcoreMesh(core_axis_name="c", subcore_axis_name="s",
                                  num_cores=_NC, num_subcores=_NS)

    @functools.partial(
        pl.kernel,
        out_type=jax.ShapeDtypeStruct((b, _LANES), jnp.float32),
        mesh=mesh,
        scratch_types=[
            pltpu.VMEM((rpw * n,), jnp.float32),
            pltpu.VMEM((_LANES,), jnp.float32),
            pltpu.SemaphoreType.DMA,
        ],
        compiler_params=pltpu.CompilerParams(needs_layout_passes=False),
    )
    def sc_reduce(x2, out, rows_v, res_v, sem):
        cid = lax.axis_index("c")
        sid = lax.axis_index("s")
        wid = sid * _NC + cid
        row0 = wid * rpw
        cps = [
            pltpu.async_copy(x2.at[row0 + rl],
                             rows_v.at[pl.ds(rl * n, n)], sem)
            for rl in range(rpw)
        ]
        lane = lax.iota(jnp.int32, _LANES)
        ivecs, wvecs = [], []
        for j in range(_KS // _LANES):
            ivv = jnp.zeros((_LANES,), jnp.int32)
            wvv = jnp.zeros((_LANES,), jnp.float32)
            for i in range(_LANES):
                ivv = jnp.where(lane == i, hist_l[j * _LANES + i], ivv)
                wvv = jnp.where(lane == i, w_l[j * _LANES + i], wvv)
            ivecs.append(ivv)
            wvecs.append(wvv)
        for rl in range(rpw):
            cps[rl].wait()
            acc = jnp.zeros((_LANES,), jnp.float32)
            for j in range(_KS // _LANES):
                vals16 = plsc.load_gather(rows_v, [ivecs[j] + (rl * n)])
                acc = acc + wvecs[j] * vals16
            dot = jnp.sum(acc)
            cur = rows_v[pl.ds(rl * n + n - _LANES, _LANES)][_LANES - 1]
            res = (cur * wsum - dot) * (1.0 / _KS)
            res_v[...] = jnp.where(lane == _LANES - 1, res, 0.0)
            pltpu.sync_copy(res_v, out.at[row0 + rl])

    return sc_reduce


def _tc_zeros_body(o_ref, z_ref, sem):
    z_ref[...] = jnp.zeros_like(z_ref)
    nblk = o_ref.shape[1] // _BW
    cps = [
        pltpu.make_async_copy(z_ref, o_ref.at[:, pl.ds(j * _BW, _BW)], sem)
        for j in range(nblk)
    ]
    for cp in cps:
        cp.start()
    for cp in cps:
        cp.wait()


def _make_tc_zeros(b: int, n: int):
    return pl.pallas_call(
        _tc_zeros_body,
        out_specs=pl.BlockSpec(memory_space=pl.ANY),
        out_shape=jax.ShapeDtypeStruct((b, n), jnp.float32),
        scratch_shapes=[
            pltpu.VMEM((b, _BW), jnp.float32),
            pltpu.SemaphoreType.DMA,
        ],
    )


def kernel(x):
    b, n = x.shape
    hist, w = _consts(n)
    upd = _make_sc_reduce(b, n, hist, w, float(w.sum()))(x)
    zeros = _make_tc_zeros(b, n)()
    return lax.dynamic_update_slice(zeros, upd, (0, n - _LANES))
